# Initial kernel scaffold; baseline (speedup 1.0000x reference)
import functools

import jax
import jax.numpy as jnp
from jax import lax
from jax.experimental import pallas as pl
from jax.experimental.pallas import tpu as pltpu
from jax.experimental.pallas import tpu_sc as plsc

_N = 10000
_E = 320000
_D = 128
_DO = 64

_NC = 2
_NS = 16
_NW = _NC * _NS
_EB = 128
_NBLK = _E // _EB
_BPW = -(-_NBLK // _NW)
_RPS = _N // _NS

_BR = 1000


def _agg_body(compute_deg, *refs):
    if compute_deg:
        (h_hbm, src_hbm, dst_hbm, znd_hbm, zdg_hbm, ones_hbm,
         out_hbm, deg_hbm,
         src_v, dst_v, rows_v, ones_v, agg_sh, deg_sh, sem) = refs
    else:
        (h_hbm, src_hbm, dst_hbm, znd_hbm,
         out_hbm,
         src_v, dst_v, rows_v, agg_sh, sem) = refs

    c = lax.axis_index("c")
    s = lax.axis_index("s")
    wid = s * _NC + c

    r0 = s * _RPS
    pltpu.sync_copy(znd_hbm.at[pl.ds(r0, _RPS)], agg_sh.at[pl.ds(r0, _RPS)])
    if compute_deg:
        pltpu.sync_copy(zdg_hbm.at[pl.ds(r0, _RPS)], deg_sh.at[pl.ds(r0, _RPS)])
        pltpu.sync_copy(ones_hbm, ones_v)
    plsc.subcore_barrier()

    b0 = wid * _BPW
    nb = jnp.minimum(_BPW, _NBLK - b0)

    def step(j, carry):
        blk = b0 + j
        pltpu.sync_copy(src_hbm.at[blk], src_v.at[0])
        pltpu.sync_copy(dst_hbm.at[blk], dst_v.at[0])
        pltpu.async_copy(h_hbm.at[src_v.at[0]], rows_v, sem).wait()
        pltpu.sync_copy(rows_v, agg_sh.at[dst_v.at[0]], add=True)
        if compute_deg:
            pltpu.sync_copy(ones_v, deg_sh.at[dst_v.at[0]], add=True)
        return carry

    lax.fori_loop(0, nb, step, 0)

    plsc.subcore_barrier()

    pltpu.sync_copy(agg_sh.at[pl.ds(r0, _RPS)],
                    out_hbm.at[c, pl.ds(r0, _RPS)])
    if compute_deg:
        pltpu.sync_copy(deg_sh.at[pl.ds(r0, _RPS)],
                        deg_hbm.at[c, pl.ds(r0, _RPS)])


def _make_agg(compute_deg):
    mesh = plsc.VectorSubcoreMesh(core_axis_name="c", subcore_axis_name="s")
    if compute_deg:
        out_type = (jax.ShapeDtypeStruct((_NC, _N, _D), jnp.float32),
                    jax.ShapeDtypeStruct((_NC, _N, 16), jnp.float32))
        scratch = [
            pltpu.VMEM((1, _EB), jnp.int32),
            pltpu.VMEM((1, _EB), jnp.int32),
            pltpu.VMEM((_EB, _D), jnp.float32),
            pltpu.VMEM((_EB, 16), jnp.float32),
            pltpu.VMEM_SHARED((_N, _D), jnp.float32),
            pltpu.VMEM_SHARED((_N, 16), jnp.float32),
            pltpu.SemaphoreType.DMA,
        ]
    else:
        out_type = jax.ShapeDtypeStruct((_NC, _N, _D), jnp.float32)
        scratch = [
            pltpu.VMEM((1, _EB), jnp.int32),
            pltpu.VMEM((1, _EB), jnp.int32),
            pltpu.VMEM((_EB, _D), jnp.float32),
            pltpu.VMEM_SHARED((_N, _D), jnp.float32),
            pltpu.SemaphoreType.DMA,
        ]
    return pl.kernel(
        functools.partial(_agg_body, compute_deg),
        out_type=out_type,
        mesh=mesh,
        scratch_types=scratch,
        name="sc_mean_agg" + ("_deg" if compute_deg else ""),
    )


def _embed_body(x_ref, w_ref, b_ref, o_ref):
    o_ref[...] = jnp.dot(x_ref[...], w_ref[...],
                         preferred_element_type=jnp.float32) + b_ref[...]


def _embed(x, W, b):
    return pl.pallas_call(
        _embed_body,
        out_shape=jax.ShapeDtypeStruct((_N, _D), jnp.float32),
        grid=(_N // _BR,),
        in_specs=[
            pl.BlockSpec((_BR, _D), lambda i: (i, 0)),
            pl.BlockSpec((_D, _D), lambda i: (0, 0)),
            pl.BlockSpec((1, _D), lambda i: (0, 0)),
        ],
        out_specs=pl.BlockSpec((_BR, _D), lambda i: (i, 0)),
    )(x, W, b)


def _norm(p_ref, dg_ref):
    a = p_ref[0] + p_ref[1]
    d = dg_ref[0, :, 0:1] + dg_ref[1, :, 0:1]
    return a * (1.0 / jnp.maximum(d, 1.0))


def _mid_body(p_ref, dg_ref, w_ref, b_ref, o_ref):
    agg = _norm(p_ref, dg_ref)
    h = jnp.dot(agg, w_ref[...], preferred_element_type=jnp.float32) + b_ref[...]
    o_ref[...] = jnp.maximum(h, 0.0)


def _mid(partials, degp, W, b):
    return pl.pallas_call(
        _mid_body,
        out_shape=jax.ShapeDtypeStruct((_N, _D), jnp.float32),
        grid=(_N // _BR,),
        in_specs=[
            pl.BlockSpec((_NC, _BR, _D), lambda i: (0, i, 0)),
            pl.BlockSpec((_NC, _BR, 16), lambda i: (0, i, 0)),
            pl.BlockSpec((_D, _D), lambda i: (0, 0)),
            pl.BlockSpec((1, _D), lambda i: (0, 0)),
        ],
        out_specs=pl.BlockSpec((_BR, _D), lambda i: (i, 0)),
    )(partials, degp, W, b)


def _final_body(p_ref, dg_ref, w2_ref, b2_ref, wo_ref, bo_ref, o_ref):
    agg = _norm(p_ref, dg_ref)
    h = jnp.dot(agg, w2_ref[...], preferred_element_type=jnp.float32) + b2_ref[...]
    h = jnp.maximum(h, 0.0)
    o_ref[...] = jnp.dot(h, wo_ref[...],
                         preferred_element_type=jnp.float32) + bo_ref[...]


def _final(partials, degp, W2, b2, Wo, bo):
    return pl.pallas_call(
        _final_body,
        out_shape=jax.ShapeDtypeStruct((_N, _DO), jnp.float32),
        grid=(_N // _BR,),
        in_specs=[
            pl.BlockSpec((_NC, _BR, _D), lambda i: (0, i, 0)),
            pl.BlockSpec((_NC, _BR, 16), lambda i: (0, i, 0)),
            pl.BlockSpec((_D, _D), lambda i: (0, 0)),
            pl.BlockSpec((1, _D), lambda i: (0, 0)),
            pl.BlockSpec((_D, _DO), lambda i: (0, 0)),
            pl.BlockSpec((1, _DO), lambda i: (0, 0)),
        ],
        out_specs=pl.BlockSpec((_BR, _DO), lambda i: (i, 0)),
    )(partials, degp, W2, b2, Wo, bo)


def kernel(x, edge_index, W_embed, b_embed, W1, b1, W2, b2, W_out, b_out):
    src = edge_index[0].reshape(_NBLK, _EB)
    dst = edge_index[1].reshape(_NBLK, _EB)
    znd = jnp.zeros((_N, _D), jnp.float32)
    zdg = jnp.zeros((_N, 16), jnp.float32)
    ones = jnp.ones((_EB, 16), jnp.float32)

    h0 = _embed(x, W_embed, b_embed.reshape(1, _D))
    partials, degp = _make_agg(True)(h0, src, dst, znd, zdg, ones)
    h1 = _mid(partials, degp, W1, b1.reshape(1, _D))
    partials2 = _make_agg(False)(h1, src, dst, znd)
    return _final(partials2, degp, W2, b2.reshape(1, _D),
                  W_out, b_out.reshape(1, _DO))


# speedup vs baseline: 5.7943x; 5.7943x over previous
import jax
import jax.numpy as jnp
from jax import lax
from jax.experimental import pallas as pl
from jax.experimental.pallas import tpu as pltpu
from jax.experimental.pallas import tpu_sc as plsc

_N = 10000
_E = 320000
_D = 128
_DO = 64

_NC = 2
_NS = 16
_NW = _NC * _NS
_EB = 128
_NBLK = _E // _EB
_BPW = -(-_NBLK // _NW)
_RSTRIDE = 624
_RSZ = 640
_NDEG = 10240

_BR = 1000


def _agg_body(h_hbm, src_hbm, dst_hbm, znd_hbm, out_hbm,
              src_v, dst_v, rows_v, agg_sh, sem):
    c = lax.axis_index("c")
    s = lax.axis_index("s")
    wid = s * _NC + c

    r0 = s * _RSTRIDE
    pltpu.sync_copy(znd_hbm.at[pl.ds(r0, _RSZ)], agg_sh.at[pl.ds(r0, _RSZ)])
    plsc.subcore_barrier()

    b0 = wid * _BPW
    nb = jnp.minimum(_BPW, _NBLK - b0)

    def step(j, carry):
        blk = b0 + j
        pltpu.sync_copy(src_hbm.at[blk, 0], src_v)
        pltpu.sync_copy(dst_hbm.at[blk, 0], dst_v)
        pltpu.async_copy(h_hbm.at[src_v], rows_v, sem).wait()
        pltpu.sync_copy(rows_v, agg_sh.at[dst_v], add=True)
        return carry

    lax.fori_loop(0, nb, step, 0)

    plsc.subcore_barrier()

    pltpu.sync_copy(agg_sh.at[pl.ds(r0, _RSZ)],
                    out_hbm.at[c, pl.ds(r0, _RSZ)])


_agg = pl.kernel(
    _agg_body,
    out_type=jax.ShapeDtypeStruct((_NC, _N, _D), jnp.float32),
    mesh=plsc.VectorSubcoreMesh(core_axis_name="c", subcore_axis_name="s"),
    scratch_types=[
        pltpu.VMEM((_EB,), jnp.int32),
        pltpu.VMEM((_EB,), jnp.int32),
        pltpu.VMEM((_EB, _D), jnp.float32),
        pltpu.VMEM_SHARED((_N, _D), jnp.float32),
        pltpu.SemaphoreType.DMA,
    ],
    name="sc_mean_agg",
)


def _deg_body(dst_hbm, znd_hbm, out_hbm, dst_v, ones_v, deg_sh):
    c = lax.axis_index("c")
    s = lax.axis_index("s")
    wid = s * _NC + c

    r0 = s * (_NDEG // _NS)
    pltpu.sync_copy(znd_hbm.at[pl.ds(r0, _NDEG // _NS)],
                    deg_sh.at[pl.ds(r0, _NDEG // _NS)])

    def fill_o(i, carry):
        for k in range(_D // 16):
            ones_v[i, pl.ds(k * 16, 16)] = jnp.ones((16,), jnp.float32)
        return carry
    lax.fori_loop(0, _EB, fill_o, 0)
    plsc.subcore_barrier()

    b0 = wid * _BPW
    nb = jnp.minimum(_BPW, _NBLK - b0)

    def step(j, carry):
        blk = b0 + j
        pltpu.sync_copy(dst_hbm.at[blk, 0], dst_v)
        pltpu.sync_copy(ones_v, deg_sh.at[dst_v], add=True)
        return carry

    lax.fori_loop(0, nb, step, 0)

    plsc.subcore_barrier()
    pltpu.sync_copy(deg_sh.at[pl.ds(r0, _NDEG // _NS)],
                    out_hbm.at[c, pl.ds(r0, _NDEG // _NS)])


_deg = pl.kernel(
    _deg_body,
    out_type=jax.ShapeDtypeStruct((_NC, _NDEG, _D), jnp.float32),
    mesh=plsc.VectorSubcoreMesh(core_axis_name="c", subcore_axis_name="s"),
    scratch_types=[
        pltpu.VMEM((_EB,), jnp.int32),
        pltpu.VMEM((_EB, _D), jnp.float32),
        pltpu.VMEM_SHARED((_NDEG, _D), jnp.float32),
    ],
    name="sc_degree",
)


def _embed_body(x_ref, w_ref, b_ref, o_ref):
    o_ref[...] = jnp.dot(x_ref[...], w_ref[...],
                         preferred_element_type=jnp.float32) + b_ref[...]


def _embed(x, W, b):
    return pl.pallas_call(
        _embed_body,
        out_shape=jax.ShapeDtypeStruct((_N, _D), jnp.float32),
        grid=(_N // _BR,),
        in_specs=[
            pl.BlockSpec((_BR, _D), lambda i: (i, 0)),
            pl.BlockSpec((_D, _D), lambda i: (0, 0)),
            pl.BlockSpec((1, _D), lambda i: (0, 0)),
        ],
        out_specs=pl.BlockSpec((_BR, _D), lambda i: (i, 0)),
    )(x, W, b)


def _norm(p_ref, dg_ref):
    a = p_ref[0] + p_ref[1]
    d = dg_ref[0, :, 0:1] + dg_ref[1, :, 0:1]
    return a * (1.0 / jnp.maximum(d, 1.0))


def _mid_body(p_ref, dg_ref, w_ref, b_ref, o_ref):
    agg = _norm(p_ref, dg_ref)
    h = jnp.dot(agg, w_ref[...], preferred_element_type=jnp.float32) + b_ref[...]
    o_ref[...] = jnp.maximum(h, 0.0)


def _mid(partials, degp, W, b):
    return pl.pallas_call(
        _mid_body,
        out_shape=jax.ShapeDtypeStruct((_N, _D), jnp.float32),
        grid=(_N // _BR,),
        in_specs=[
            pl.BlockSpec((_NC, _BR, _D), lambda i: (0, i, 0)),
            pl.BlockSpec((_NC, _BR, _D), lambda i: (0, i, 0)),
            pl.BlockSpec((_D, _D), lambda i: (0, 0)),
            pl.BlockSpec((1, _D), lambda i: (0, 0)),
        ],
        out_specs=pl.BlockSpec((_BR, _D), lambda i: (i, 0)),
    )(partials, degp, W, b)


def _final_body(p_ref, dg_ref, w2_ref, b2_ref, wo_ref, bo_ref, o_ref):
    agg = _norm(p_ref, dg_ref)
    h = jnp.dot(agg, w2_ref[...], preferred_element_type=jnp.float32) + b2_ref[...]
    h = jnp.maximum(h, 0.0)
    o_ref[...] = jnp.dot(h, wo_ref[...],
                         preferred_element_type=jnp.float32) + bo_ref[...]


def _final(partials, degp, W2, b2, Wo, bo):
    return pl.pallas_call(
        _final_body,
        out_shape=jax.ShapeDtypeStruct((_N, _DO), jnp.float32),
        grid=(_N // _BR,),
        in_specs=[
            pl.BlockSpec((_NC, _BR, _D), lambda i: (0, i, 0)),
            pl.BlockSpec((_NC, _BR, _D), lambda i: (0, i, 0)),
            pl.BlockSpec((_D, _D), lambda i: (0, 0)),
            pl.BlockSpec((1, _D), lambda i: (0, 0)),
            pl.BlockSpec((_D, _DO), lambda i: (0, 0)),
            pl.BlockSpec((1, _DO), lambda i: (0, 0)),
        ],
        out_specs=pl.BlockSpec((_BR, _DO), lambda i: (i, 0)),
    )(partials, degp, W2, b2, Wo, bo)


def kernel(x, edge_index, W_embed, b_embed, W1, b1, W2, b2, W_out, b_out):
    src = edge_index[0].reshape(_NBLK, 1, _EB)
    dst = edge_index[1].reshape(_NBLK, 1, _EB)
    znd = jnp.zeros((_NDEG, _D), jnp.float32)

    h0 = _embed(x, W_embed, b_embed.reshape(1, _D))
    degp = _deg(dst, znd)
    partials = _agg(h0, src, dst, znd)
    h1 = _mid(partials, degp, W1, b1.reshape(1, _D))
    partials2 = _agg(h1, src, dst, znd)
    return _final(partials2, degp, W2, b2.reshape(1, _D),
                  W_out, b_out.reshape(1, _DO))


# speedup vs baseline: 6.5387x; 1.1285x over previous
import jax
import jax.numpy as jnp
from jax import lax
from jax.experimental import pallas as pl
from jax.experimental.pallas import tpu as pltpu
from jax.experimental.pallas import tpu_sc as plsc

_N = 10000
_E = 320000
_D = 128
_DO = 64

_NC = 2
_NS = 16
_NW = _NC * _NS
_EB = 128
_NBLK = _E // _EB
_BPW = -(-_NBLK // _NW)
_RSTRIDE = 624
_RSZ = 640
_NDEG = 10240

_BR = 1000


_HB = _EB // 2


def _agg_body(h_hbm, src_hbm, dst_hbm, znd_hbm, out_hbm,
              src_v, dst_v, dst_a, dst_b, rows_a, rows_b,
              agg_sh, sem_i, sem_a, sem_b):
    c = lax.axis_index("c")
    s = lax.axis_index("s")
    wid = s * _NC + c

    r0 = s * _RSTRIDE
    pltpu.sync_copy(znd_hbm.at[pl.ds(r0, _RSZ)], agg_sh.at[pl.ds(r0, _RSZ)])
    plsc.subcore_barrier()

    b0 = wid * _BPW
    nb = jnp.minimum(_BPW, _NBLK - b0)

    def step(j, carry):
        blk = b0 + j
        di = pltpu.async_copy(src_hbm.at[blk, 0], src_v, sem_i)
        dj = pltpu.async_copy(dst_hbm.at[blk, 0], dst_v, sem_i)
        di.wait()
        dj.wait()
        for k in range(_HB // 16):
            dst_a[pl.ds(k * 16, 16)] = dst_v[pl.ds(k * 16, 16)]
            dst_b[pl.ds(k * 16, 16)] = dst_v[pl.ds(_HB + k * 16, 16)]
        ga = pltpu.async_copy(h_hbm.at[src_v.at[pl.ds(0, _HB)]], rows_a, sem_a)
        gb = pltpu.async_copy(h_hbm.at[src_v.at[pl.ds(_HB, _HB)]], rows_b, sem_b)
        ga.wait()
        pltpu.sync_copy(rows_a, agg_sh.at[dst_a], add=True)
        gb.wait()
        pltpu.sync_copy(rows_b, agg_sh.at[dst_b], add=True)
        return carry

    lax.fori_loop(0, nb, step, 0)

    plsc.subcore_barrier()

    pltpu.sync_copy(agg_sh.at[pl.ds(r0, _RSZ)],
                    out_hbm.at[c, pl.ds(r0, _RSZ)])


_agg = pl.kernel(
    _agg_body,
    out_type=jax.ShapeDtypeStruct((_NC, _N, _D), jnp.float32),
    mesh=plsc.VectorSubcoreMesh(core_axis_name="c", subcore_axis_name="s"),
    scratch_types=[
        pltpu.VMEM((_EB,), jnp.int32),
        pltpu.VMEM((_EB,), jnp.int32),
        pltpu.VMEM((_HB,), jnp.int32),
        pltpu.VMEM((_HB,), jnp.int32),
        pltpu.VMEM((_HB, _D), jnp.float32),
        pltpu.VMEM((_HB, _D), jnp.float32),
        pltpu.VMEM_SHARED((_N, _D), jnp.float32),
        pltpu.SemaphoreType.DMA,
        pltpu.SemaphoreType.DMA,
        pltpu.SemaphoreType.DMA,
    ],
    name="sc_mean_agg",
)


def _deg_body(dst_hbm, znd_hbm, out_hbm, dst_v, ones_v, deg_sh):
    c = lax.axis_index("c")
    s = lax.axis_index("s")
    wid = s * _NC + c

    r0 = s * (_NDEG // _NS)
    pltpu.sync_copy(znd_hbm.at[pl.ds(r0, _NDEG // _NS)],
                    deg_sh.at[pl.ds(r0, _NDEG // _NS)])

    def fill_o(i, carry):
        for k in range(_D // 16):
            ones_v[i, pl.ds(k * 16, 16)] = jnp.ones((16,), jnp.float32)
        return carry
    lax.fori_loop(0, _EB, fill_o, 0)
    plsc.subcore_barrier()

    b0 = wid * _BPW
    nb = jnp.minimum(_BPW, _NBLK - b0)

    def step(j, carry):
        blk = b0 + j
        pltpu.sync_copy(dst_hbm.at[blk, 0], dst_v)
        pltpu.sync_copy(ones_v, deg_sh.at[dst_v], add=True)
        return carry

    lax.fori_loop(0, nb, step, 0)

    plsc.subcore_barrier()
    pltpu.sync_copy(deg_sh.at[pl.ds(r0, _NDEG // _NS)],
                    out_hbm.at[c, pl.ds(r0, _NDEG // _NS)])


_deg = pl.kernel(
    _deg_body,
    out_type=jax.ShapeDtypeStruct((_NC, _NDEG, _D), jnp.float32),
    mesh=plsc.VectorSubcoreMesh(core_axis_name="c", subcore_axis_name="s"),
    scratch_types=[
        pltpu.VMEM((_EB,), jnp.int32),
        pltpu.VMEM((_EB, _D), jnp.float32),
        pltpu.VMEM_SHARED((_NDEG, _D), jnp.float32),
    ],
    name="sc_degree",
)


def _embed_body(x_ref, w_ref, b_ref, o_ref):
    o_ref[...] = jnp.dot(x_ref[...], w_ref[...],
                         preferred_element_type=jnp.float32) + b_ref[...]


def _embed(x, W, b):
    return pl.pallas_call(
        _embed_body,
        out_shape=jax.ShapeDtypeStruct((_N, _D), jnp.float32),
        grid=(_N // _BR,),
        in_specs=[
            pl.BlockSpec((_BR, _D), lambda i: (i, 0)),
            pl.BlockSpec((_D, _D), lambda i: (0, 0)),
            pl.BlockSpec((1, _D), lambda i: (0, 0)),
        ],
        out_specs=pl.BlockSpec((_BR, _D), lambda i: (i, 0)),
    )(x, W, b)


def _norm(p_ref, dg_ref):
    a = p_ref[0] + p_ref[1]
    d = dg_ref[0, :, 0:1] + dg_ref[1, :, 0:1]
    return a * (1.0 / jnp.maximum(d, 1.0))


def _mid_body(p_ref, dg_ref, w_ref, b_ref, o_ref):
    agg = _norm(p_ref, dg_ref)
    h = jnp.dot(agg, w_ref[...], preferred_element_type=jnp.float32) + b_ref[...]
    o_ref[...] = jnp.maximum(h, 0.0)


def _mid(partials, degp, W, b):
    return pl.pallas_call(
        _mid_body,
        out_shape=jax.ShapeDtypeStruct((_N, _D), jnp.float32),
        grid=(_N // _BR,),
        in_specs=[
            pl.BlockSpec((_NC, _BR, _D), lambda i: (0, i, 0)),
            pl.BlockSpec((_NC, _BR, _D), lambda i: (0, i, 0)),
            pl.BlockSpec((_D, _D), lambda i: (0, 0)),
            pl.BlockSpec((1, _D), lambda i: (0, 0)),
        ],
        out_specs=pl.BlockSpec((_BR, _D), lambda i: (i, 0)),
    )(partials, degp, W, b)


def _final_body(p_ref, dg_ref, w2_ref, b2_ref, wo_ref, bo_ref, o_ref):
    agg = _norm(p_ref, dg_ref)
    h = jnp.dot(agg, w2_ref[...], preferred_element_type=jnp.float32) + b2_ref[...]
    h = jnp.maximum(h, 0.0)
    o_ref[...] = jnp.dot(h, wo_ref[...],
                         preferred_element_type=jnp.float32) + bo_ref[...]


def _final(partials, degp, W2, b2, Wo, bo):
    return pl.pallas_call(
        _final_body,
        out_shape=jax.ShapeDtypeStruct((_N, _DO), jnp.float32),
        grid=(_N // _BR,),
        in_specs=[
            pl.BlockSpec((_NC, _BR, _D), lambda i: (0, i, 0)),
            pl.BlockSpec((_NC, _BR, _D), lambda i: (0, i, 0)),
            pl.BlockSpec((_D, _D), lambda i: (0, 0)),
            pl.BlockSpec((1, _D), lambda i: (0, 0)),
            pl.BlockSpec((_D, _DO), lambda i: (0, 0)),
            pl.BlockSpec((1, _DO), lambda i: (0, 0)),
        ],
        out_specs=pl.BlockSpec((_BR, _DO), lambda i: (i, 0)),
    )(partials, degp, W2, b2, Wo, bo)


def kernel(x, edge_index, W_embed, b_embed, W1, b1, W2, b2, W_out, b_out):
    src = edge_index[0].reshape(_NBLK, 1, _EB)
    dst = edge_index[1].reshape(_NBLK, 1, _EB)
    znd = jnp.zeros((_NDEG, _D), jnp.float32)

    h0 = _embed(x, W_embed, b_embed.reshape(1, _D))
    degp = _deg(dst, znd)
    partials = _agg(h0, src, dst, znd)
    h1 = _mid(partials, degp, W1, b1.reshape(1, _D))
    partials2 = _agg(h1, src, dst, znd)
    return _final(partials2, degp, W2, b2.reshape(1, _D),
                  W_out, b_out.reshape(1, _DO))


# speedup vs baseline: 7.4749x; 1.1432x over previous
import jax
import jax.numpy as jnp
from jax import lax
from jax.experimental import pallas as pl
from jax.experimental.pallas import tpu as pltpu
from jax.experimental.pallas import tpu_sc as plsc

_N = 10000
_E = 320000
_D = 128
_DO = 64

_NC = 2
_NS = 16
_NW = _NC * _NS
_EB = 128
_NBLK = _E // _EB
_BPW = -(-_NBLK // _NW)
_RSTRIDE = 624
_RSZ = 640
_NDEG = 10240

_BR = 1000


_HB = _EB // 2


def _agg_body(h_hbm, src_hbm, dst_hbm, znd_hbm, out_hbm,
              src_v, dst_v, src_w, dst_w, dst_a, dst_b, rows_a, rows_b,
              agg_sh, sem_i, sem_j, sem_a, sem_b):
    c = lax.axis_index("c")
    s = lax.axis_index("s")
    wid = s * _NC + c

    r0 = s * _RSTRIDE
    pltpu.sync_copy(znd_hbm.at[pl.ds(r0, _RSZ)], agg_sh.at[pl.ds(r0, _RSZ)])
    plsc.subcore_barrier()

    b0 = wid * _BPW
    nb = jnp.minimum(_BPW, _NBLK - b0)
    last = _NBLK - 1

    def fetch(blk, sv, dv, sem):
        pltpu.async_copy(src_hbm.at[blk, 0], sv, sem)
        pltpu.async_copy(dst_hbm.at[blk, 0], dv, sem)

    def wait_fetch(blk, sv, dv, sem):
        pltpu.make_async_copy(src_hbm.at[blk, 0], sv, sem).wait()
        pltpu.make_async_copy(dst_hbm.at[blk, 0], dv, sem).wait()

    def process(sv, dv):
        for k in range(_HB // 16):
            dst_a[pl.ds(k * 16, 16)] = dv[pl.ds(k * 16, 16)]
            dst_b[pl.ds(k * 16, 16)] = dv[pl.ds(_HB + k * 16, 16)]
        ga = pltpu.async_copy(h_hbm.at[sv.at[pl.ds(0, _HB)]], rows_a, sem_a)
        gb = pltpu.async_copy(h_hbm.at[sv.at[pl.ds(_HB, _HB)]], rows_b, sem_b)
        ga.wait()
        pltpu.sync_copy(rows_a, agg_sh.at[dst_a], add=True)
        gb.wait()
        pltpu.sync_copy(rows_b, agg_sh.at[dst_b], add=True)

    pairs = nb // 2
    tail = nb - 2 * pairs

    fetch(b0, src_v, dst_v, sem_i)

    def step(jj, carry):
        blk0 = b0 + 2 * jj
        blk1 = jnp.minimum(blk0 + 1, last)
        nxt = jnp.minimum(blk0 + 2, last)
        fetch(blk1, src_w, dst_w, sem_j)
        wait_fetch(blk0, src_v, dst_v, sem_i)
        process(src_v, dst_v)
        fetch(nxt, src_v, dst_v, sem_i)
        wait_fetch(blk1, src_w, dst_w, sem_j)
        process(src_w, dst_w)
        return carry

    lax.fori_loop(0, pairs, step, 0)

    tb = jnp.minimum(b0 + 2 * pairs, last)
    wait_fetch(tb, src_v, dst_v, sem_i)

    @pl.when(tail == 1)
    def _():
        process(src_v, dst_v)

    plsc.subcore_barrier()

    pltpu.sync_copy(agg_sh.at[pl.ds(r0, _RSZ)],
                    out_hbm.at[c, pl.ds(r0, _RSZ)])


_agg = pl.kernel(
    _agg_body,
    out_type=jax.ShapeDtypeStruct((_NC, _N, _D), jnp.float32),
    mesh=plsc.VectorSubcoreMesh(core_axis_name="c", subcore_axis_name="s"),
    scratch_types=[
        pltpu.VMEM((_EB,), jnp.int32),
        pltpu.VMEM((_EB,), jnp.int32),
        pltpu.VMEM((_EB,), jnp.int32),
        pltpu.VMEM((_EB,), jnp.int32),
        pltpu.VMEM((_HB,), jnp.int32),
        pltpu.VMEM((_HB,), jnp.int32),
        pltpu.VMEM((_HB, _D), jnp.float32),
        pltpu.VMEM((_HB, _D), jnp.float32),
        pltpu.VMEM_SHARED((_N, _D), jnp.float32),
        pltpu.SemaphoreType.DMA,
        pltpu.SemaphoreType.DMA,
        pltpu.SemaphoreType.DMA,
        pltpu.SemaphoreType.DMA,
    ],
    name="sc_mean_agg",
)


def _deg_body(dst_hbm, znd_hbm, out_hbm, dst_v, ones_v, deg_sh):
    c = lax.axis_index("c")
    s = lax.axis_index("s")
    wid = s * _NC + c

    r0 = s * (_NDEG // _NS)
    pltpu.sync_copy(znd_hbm.at[pl.ds(r0, _NDEG // _NS)],
                    deg_sh.at[pl.ds(r0, _NDEG // _NS)])

    def fill_o(i, carry):
        for k in range(_D // 16):
            ones_v[i, pl.ds(k * 16, 16)] = jnp.ones((16,), jnp.float32)
        return carry
    lax.fori_loop(0, _EB, fill_o, 0)
    plsc.subcore_barrier()

    b0 = wid * _BPW
    nb = jnp.minimum(_BPW, _NBLK - b0)

    def step(j, carry):
        blk = b0 + j
        pltpu.sync_copy(dst_hbm.at[blk, 0], dst_v)
        pltpu.sync_copy(ones_v, deg_sh.at[dst_v], add=True)
        return carry

    lax.fori_loop(0, nb, step, 0)

    plsc.subcore_barrier()
    pltpu.sync_copy(deg_sh.at[pl.ds(r0, _NDEG // _NS)],
                    out_hbm.at[c, pl.ds(r0, _NDEG // _NS)])


_deg = pl.kernel(
    _deg_body,
    out_type=jax.ShapeDtypeStruct((_NC, _NDEG, _D), jnp.float32),
    mesh=plsc.VectorSubcoreMesh(core_axis_name="c", subcore_axis_name="s"),
    scratch_types=[
        pltpu.VMEM((_EB,), jnp.int32),
        pltpu.VMEM((_EB, _D), jnp.float32),
        pltpu.VMEM_SHARED((_NDEG, _D), jnp.float32),
    ],
    name="sc_degree",
)


def _embed_body(x_ref, w_ref, b_ref, o_ref):
    o_ref[...] = jnp.dot(x_ref[...], w_ref[...],
                         preferred_element_type=jnp.float32) + b_ref[...]


def _embed(x, W, b):
    return pl.pallas_call(
        _embed_body,
        out_shape=jax.ShapeDtypeStruct((_N, _D), jnp.float32),
        grid=(_N // _BR,),
        in_specs=[
            pl.BlockSpec((_BR, _D), lambda i: (i, 0)),
            pl.BlockSpec((_D, _D), lambda i: (0, 0)),
            pl.BlockSpec((1, _D), lambda i: (0, 0)),
        ],
        out_specs=pl.BlockSpec((_BR, _D), lambda i: (i, 0)),
    )(x, W, b)


def _norm(p_ref, dg_ref):
    a = p_ref[0] + p_ref[1]
    d = dg_ref[0, :, 0:1] + dg_ref[1, :, 0:1]
    return a * (1.0 / jnp.maximum(d, 1.0))


def _mid_body(p_ref, dg_ref, w_ref, b_ref, o_ref):
    agg = _norm(p_ref, dg_ref)
    h = jnp.dot(agg, w_ref[...], preferred_element_type=jnp.float32) + b_ref[...]
    o_ref[...] = jnp.maximum(h, 0.0)


def _mid(partials, degp, W, b):
    return pl.pallas_call(
        _mid_body,
        out_shape=jax.ShapeDtypeStruct((_N, _D), jnp.float32),
        grid=(_N // _BR,),
        in_specs=[
            pl.BlockSpec((_NC, _BR, _D), lambda i: (0, i, 0)),
            pl.BlockSpec((_NC, _BR, _D), lambda i: (0, i, 0)),
            pl.BlockSpec((_D, _D), lambda i: (0, 0)),
            pl.BlockSpec((1, _D), lambda i: (0, 0)),
        ],
        out_specs=pl.BlockSpec((_BR, _D), lambda i: (i, 0)),
    )(partials, degp, W, b)


def _final_body(p_ref, dg_ref, w2_ref, b2_ref, wo_ref, bo_ref, o_ref):
    agg = _norm(p_ref, dg_ref)
    h = jnp.dot(agg, w2_ref[...], preferred_element_type=jnp.float32) + b2_ref[...]
    h = jnp.maximum(h, 0.0)
    o_ref[...] = jnp.dot(h, wo_ref[...],
                         preferred_element_type=jnp.float32) + bo_ref[...]


def _final(partials, degp, W2, b2, Wo, bo):
    return pl.pallas_call(
        _final_body,
        out_shape=jax.ShapeDtypeStruct((_N, _DO), jnp.float32),
        grid=(_N // _BR,),
        in_specs=[
            pl.BlockSpec((_NC, _BR, _D), lambda i: (0, i, 0)),
            pl.BlockSpec((_NC, _BR, _D), lambda i: (0, i, 0)),
            pl.BlockSpec((_D, _D), lambda i: (0, 0)),
            pl.BlockSpec((1, _D), lambda i: (0, 0)),
            pl.BlockSpec((_D, _DO), lambda i: (0, 0)),
            pl.BlockSpec((1, _DO), lambda i: (0, 0)),
        ],
        out_specs=pl.BlockSpec((_BR, _DO), lambda i: (i, 0)),
    )(partials, degp, W2, b2, Wo, bo)


def kernel(x, edge_index, W_embed, b_embed, W1, b1, W2, b2, W_out, b_out):
    src = edge_index[0].reshape(_NBLK, 1, _EB)
    dst = edge_index[1].reshape(_NBLK, 1, _EB)
    znd = jnp.zeros((_NDEG, _D), jnp.float32)

    h0 = _embed(x, W_embed, b_embed.reshape(1, _D))
    degp = _deg(dst, znd)
    partials = _agg(h0, src, dst, znd)
    h1 = _mid(partials, degp, W1, b1.reshape(1, _D))
    partials2 = _agg(h1, src, dst, znd)
    return _final(partials2, degp, W2, b2.reshape(1, _D),
                  W_out, b_out.reshape(1, _DO))


# speedup vs baseline: 8.1081x; 1.0847x over previous
import jax
import jax.numpy as jnp
from jax import lax
from jax.experimental import pallas as pl
from jax.experimental.pallas import tpu as pltpu
from jax.experimental.pallas import tpu_sc as plsc

_N = 10000
_E = 320000
_D = 128
_DO = 64

_NC = 2
_NS = 16
_NW = _NC * _NS
_EB = 128
_NBLK = _E // _EB
_BPW = -(-_NBLK // _NW)
_RSTRIDE = 624
_RSZ = 640
_NDEG = 10240

_BR = 1000


_HB = _EB // 2


def _agg_body(h_hbm, src_hbm, dst_hbm, znd_hbm, out_hbm,
              src_v, dst_v, src_w, dst_w, dst_a, dst_b, rows_a, rows_b,
              agg_sh, sem_i, sem_j, sem_a, sem_b):
    c = lax.axis_index("c")
    s = lax.axis_index("s")
    wid = s * _NC + c

    r0 = s * _RSTRIDE
    pltpu.sync_copy(znd_hbm.at[pl.ds(r0, _RSZ)], agg_sh.at[pl.ds(r0, _RSZ)])
    plsc.subcore_barrier()

    b0 = wid * _BPW
    nb = jnp.minimum(_BPW, _NBLK - b0)
    last = _NBLK - 1

    def fetch(blk, sv, dv, sem):
        pltpu.async_copy(src_hbm.at[blk, 0], sv, sem)
        pltpu.async_copy(dst_hbm.at[blk, 0], dv, sem)

    def wait_fetch(blk, sv, dv, sem):
        pltpu.make_async_copy(src_hbm.at[blk, 0], sv, sem).wait()
        pltpu.make_async_copy(dst_hbm.at[blk, 0], dv, sem).wait()

    def process(sv, dv):
        for k in range(_HB // 16):
            dst_a[pl.ds(k * 16, 16)] = dv[pl.ds(k * 16, 16)]
            dst_b[pl.ds(k * 16, 16)] = dv[pl.ds(_HB + k * 16, 16)]
        ga = pltpu.async_copy(h_hbm.at[sv.at[pl.ds(0, _HB)]], rows_a, sem_a)
        gb = pltpu.async_copy(h_hbm.at[sv.at[pl.ds(_HB, _HB)]], rows_b, sem_b)
        ga.wait()
        pltpu.sync_copy(rows_a, agg_sh.at[dst_a], add=True)
        gb.wait()
        pltpu.sync_copy(rows_b, agg_sh.at[dst_b], add=True)

    pairs = nb // 2
    tail = nb - 2 * pairs

    fetch(b0, src_v, dst_v, sem_i)

    def step(jj, carry):
        blk0 = b0 + 2 * jj
        blk1 = jnp.minimum(blk0 + 1, last)
        nxt = jnp.minimum(blk0 + 2, last)
        fetch(blk1, src_w, dst_w, sem_j)
        wait_fetch(blk0, src_v, dst_v, sem_i)
        process(src_v, dst_v)
        fetch(nxt, src_v, dst_v, sem_i)
        wait_fetch(blk1, src_w, dst_w, sem_j)
        process(src_w, dst_w)
        return carry

    lax.fori_loop(0, pairs, step, 0)

    tb = jnp.minimum(b0 + 2 * pairs, last)
    wait_fetch(tb, src_v, dst_v, sem_i)

    @pl.when(tail == 1)
    def _():
        process(src_v, dst_v)

    plsc.subcore_barrier()

    pltpu.sync_copy(agg_sh.at[pl.ds(r0, _RSZ)],
                    out_hbm.at[c, pl.ds(r0, _RSZ)])


_agg = pl.kernel(
    _agg_body,
    out_type=jax.ShapeDtypeStruct((_NC, _N, _D), jnp.float32),
    mesh=plsc.VectorSubcoreMesh(core_axis_name="c", subcore_axis_name="s"),
    scratch_types=[
        pltpu.VMEM((_EB,), jnp.int32),
        pltpu.VMEM((_EB,), jnp.int32),
        pltpu.VMEM((_EB,), jnp.int32),
        pltpu.VMEM((_EB,), jnp.int32),
        pltpu.VMEM((_HB,), jnp.int32),
        pltpu.VMEM((_HB,), jnp.int32),
        pltpu.VMEM((_HB, _D), jnp.float32),
        pltpu.VMEM((_HB, _D), jnp.float32),
        pltpu.VMEM_SHARED((_N, _D), jnp.float32),
        pltpu.SemaphoreType.DMA,
        pltpu.SemaphoreType.DMA,
        pltpu.SemaphoreType.DMA,
        pltpu.SemaphoreType.DMA,
    ],
    name="sc_mean_agg",
)


def _deg_body(dst_hbm, znd_hbm, out_hbm, dst_v, dst_w, ones_v, deg_sh,
              sem_i, sem_j):
    c = lax.axis_index("c")
    s = lax.axis_index("s")
    wid = s * _NC + c

    r0 = s * (_NDEG // _NS)
    pltpu.sync_copy(znd_hbm.at[pl.ds(r0, _NDEG // _NS)],
                    deg_sh.at[pl.ds(r0, _NDEG // _NS)])

    def fill_o(i, carry):
        for k in range(_D // 16):
            ones_v[i, pl.ds(k * 16, 16)] = jnp.ones((16,), jnp.float32)
        return carry
    lax.fori_loop(0, _EB, fill_o, 0)
    plsc.subcore_barrier()

    b0 = wid * _BPW
    nb = jnp.minimum(_BPW, _NBLK - b0)
    last = _NBLK - 1
    pairs = nb // 2
    tail = nb - 2 * pairs

    pltpu.async_copy(dst_hbm.at[b0, 0], dst_v, sem_i)

    def step(jj, carry):
        blk0 = b0 + 2 * jj
        blk1 = jnp.minimum(blk0 + 1, last)
        nxt = jnp.minimum(blk0 + 2, last)
        pltpu.async_copy(dst_hbm.at[blk1, 0], dst_w, sem_j)
        pltpu.make_async_copy(dst_hbm.at[blk0, 0], dst_v, sem_i).wait()
        pltpu.sync_copy(ones_v, deg_sh.at[dst_v], add=True)
        pltpu.async_copy(dst_hbm.at[nxt, 0], dst_v, sem_i)
        pltpu.make_async_copy(dst_hbm.at[blk1, 0], dst_w, sem_j).wait()
        pltpu.sync_copy(ones_v, deg_sh.at[dst_w], add=True)
        return carry

    lax.fori_loop(0, pairs, step, 0)

    tb = jnp.minimum(b0 + 2 * pairs, last)
    pltpu.make_async_copy(dst_hbm.at[tb, 0], dst_v, sem_i).wait()

    @pl.when(tail == 1)
    def _():
        pltpu.sync_copy(ones_v, deg_sh.at[dst_v], add=True)

    plsc.subcore_barrier()
    pltpu.sync_copy(deg_sh.at[pl.ds(r0, _NDEG // _NS)],
                    out_hbm.at[c, pl.ds(r0, _NDEG // _NS)])


_deg = pl.kernel(
    _deg_body,
    out_type=jax.ShapeDtypeStruct((_NC, _NDEG, _D), jnp.float32),
    mesh=plsc.VectorSubcoreMesh(core_axis_name="c", subcore_axis_name="s"),
    scratch_types=[
        pltpu.VMEM((_EB,), jnp.int32),
        pltpu.VMEM((_EB,), jnp.int32),
        pltpu.VMEM((_EB, _D), jnp.float32),
        pltpu.VMEM_SHARED((_NDEG, _D), jnp.float32),
        pltpu.SemaphoreType.DMA,
        pltpu.SemaphoreType.DMA,
    ],
    name="sc_degree",
)


def _norm(p_ref, dg_ref):
    a = p_ref[0] + p_ref[1]
    d = dg_ref[0, :, 0:1] + dg_ref[1, :, 0:1]
    return a * (1.0 / jnp.maximum(d, 1.0))


def _mid_body(p_ref, dg_ref, we_ref, be_ref, w_ref, b_ref, o_ref):
    a = _norm(p_ref, dg_ref)
    d = dg_ref[0, :, 0:1] + dg_ref[1, :, 0:1]
    mask = jnp.where(d > 0.0, 1.0, 0.0)
    g = jnp.dot(a, we_ref[...], preferred_element_type=jnp.float32)
    g = g + be_ref[...] * mask
    h = jnp.dot(g, w_ref[...], preferred_element_type=jnp.float32) + b_ref[...]
    o_ref[...] = jnp.maximum(h, 0.0)


def _mid(partials, degp, We, be, W, b):
    return pl.pallas_call(
        _mid_body,
        out_shape=jax.ShapeDtypeStruct((_N, _D), jnp.float32),
        grid=(_N // _BR,),
        in_specs=[
            pl.BlockSpec((_NC, _BR, _D), lambda i: (0, i, 0)),
            pl.BlockSpec((_NC, _BR, _D), lambda i: (0, i, 0)),
            pl.BlockSpec((_D, _D), lambda i: (0, 0)),
            pl.BlockSpec((1, _D), lambda i: (0, 0)),
            pl.BlockSpec((_D, _D), lambda i: (0, 0)),
            pl.BlockSpec((1, _D), lambda i: (0, 0)),
        ],
        out_specs=pl.BlockSpec((_BR, _D), lambda i: (i, 0)),
    )(partials, degp, We, be, W, b)


def _final_body(p_ref, dg_ref, w2_ref, b2_ref, wo_ref, bo_ref, o_ref):
    agg = _norm(p_ref, dg_ref)
    h = jnp.dot(agg, w2_ref[...], preferred_element_type=jnp.float32) + b2_ref[...]
    h = jnp.maximum(h, 0.0)
    o_ref[...] = jnp.dot(h, wo_ref[...],
                         preferred_element_type=jnp.float32) + bo_ref[...]


def _final(partials, degp, W2, b2, Wo, bo):
    return pl.pallas_call(
        _final_body,
        out_shape=jax.ShapeDtypeStruct((_N, _DO), jnp.float32),
        grid=(_N // _BR,),
        in_specs=[
            pl.BlockSpec((_NC, _BR, _D), lambda i: (0, i, 0)),
            pl.BlockSpec((_NC, _BR, _D), lambda i: (0, i, 0)),
            pl.BlockSpec((_D, _D), lambda i: (0, 0)),
            pl.BlockSpec((1, _D), lambda i: (0, 0)),
            pl.BlockSpec((_D, _DO), lambda i: (0, 0)),
            pl.BlockSpec((1, _DO), lambda i: (0, 0)),
        ],
        out_specs=pl.BlockSpec((_BR, _DO), lambda i: (i, 0)),
    )(partials, degp, W2, b2, Wo, bo)


def kernel(x, edge_index, W_embed, b_embed, W1, b1, W2, b2, W_out, b_out):
    src = edge_index[0].reshape(_NBLK, 1, _EB)
    dst = edge_index[1].reshape(_NBLK, 1, _EB)
    znd = jnp.zeros((_NDEG, _D), jnp.float32)

    degp = _deg(dst, znd)
    partials = _agg(x, src, dst, znd)
    h1 = _mid(partials, degp, W_embed, b_embed.reshape(1, _D),
              W1, b1.reshape(1, _D))
    partials2 = _agg(h1, src, dst, znd)
    return _final(partials2, degp, W2, b2.reshape(1, _D),
                  W_out, b_out.reshape(1, _DO))
